# baseline (speedup 1.0000x reference)
import jax
import jax.numpy as jnp
import numpy as np
from jax.experimental import pallas as pl
from jax.experimental.pallas import tpu as pltpu

V = 100000
ROWS = 128
RG = 2
RB = ROWS // RG
CB = 8192
NCB = (V + CB - 1) // CB

_K0 = 0
_K1 = 42
_KS2 = _K0 ^ _K1 ^ 0x1BD11BDA

_TINY = np.float32(1.1754944e-38)
_NEG_INF = np.float32(float("-inf"))
_BIG_IDX = np.int32(0x7FFFFFFF)


def _threefry2x32_bits(cnt):
    u32 = jnp.uint32
    rot = lambda v, r: (v << u32(r)) | (v >> u32(32 - r))
    ks = (u32(_K0), u32(_K1), u32(_KS2))
    rotations = ((13, 15, 26, 6), (17, 29, 16, 24))
    inject = ((1, 2), (2, 0), (0, 1), (1, 2), (2, 0))

    x0 = jnp.full(cnt.shape, ks[0], u32)
    x1 = cnt + ks[1]
    for g in range(5):
        for r in rotations[g % 2]:
            x0 = x0 + x1
            x1 = rot(x1, r)
            x1 = x1 ^ x0
        a, b = inject[g]
        x0 = x0 + ks[a]
        x1 = x1 + ks[b] + u32(g + 1)
    return x0 ^ x1


def _gumbel_from_bits(bits):
    float_bits = (bits >> jnp.uint32(9)) | jnp.uint32(0x3F800000)
    f = jax.lax.bitcast_convert_type(float_bits, jnp.float32) - jnp.float32(1.0)
    one = jnp.float32(1.0)
    u = jnp.maximum(_TINY, f * (one - _TINY) + _TINY)
    return -jnp.log(-jnp.log(u))


def _block_argmax(vals, cols):
    m = jnp.max(vals, axis=1, keepdims=True)
    idx = jnp.min(jnp.where(vals == m, cols, _BIG_IDX), axis=1, keepdims=True)
    return m, idx


def _sampler_kernel(logits_ref, temps_ref, out_ref,
                    gmax, gidx, smax, sidx):
    g = pl.program_id(0)
    j = pl.program_id(1)

    blk = logits_ref[...]
    t = temps_ref[...]
    safe_t = jnp.where(t == 0.0, jnp.float32(1.0), t)

    row = g * RB + jax.lax.broadcasted_iota(jnp.int32, (RB, CB), 0)
    col = j * CB + jax.lax.broadcasted_iota(jnp.int32, (RB, CB), 1)
    valid = col < V

    cnt = (row * V + col).astype(jnp.uint32)
    gum = _gumbel_from_bits(_threefry2x32_bits(cnt))

    sval = jnp.where(valid, blk / safe_t + gum, _NEG_INF)
    gval = jnp.where(valid, blk, _NEG_INF)

    bgm, bgi = _block_argmax(gval, col)
    bsm, bsi = _block_argmax(sval, col)

    @pl.when(j == 0)
    def _init():
        gmax[...] = bgm
        gidx[...] = bgi
        smax[...] = bsm
        sidx[...] = bsi

    @pl.when(j > 0)
    def _update():
        gu = bgm > gmax[...]
        gidx[...] = jnp.where(gu, bgi, gidx[...])
        gmax[...] = jnp.where(gu, bgm, gmax[...])
        su = bsm > smax[...]
        sidx[...] = jnp.where(su, bsi, sidx[...])
        smax[...] = jnp.where(su, bsm, smax[...])

    @pl.when(j == NCB - 1)
    def _emit():
        out_ref[...] = jnp.where(t == 0.0, gidx[...], sidx[...])


def kernel(logits, temperatures):
    logits = logits.astype(jnp.float32)
    temps2d = temperatures.reshape(ROWS, 1)
    out = pl.pallas_call(
        _sampler_kernel,
        grid=(RG, NCB),
        in_specs=[
            pl.BlockSpec((RB, CB), lambda g, j: (g, j)),
            pl.BlockSpec((RB, 1), lambda g, j: (g, 0)),
        ],
        out_specs=pl.BlockSpec((RB, 1), lambda g, j: (g, 0)),
        out_shape=jax.ShapeDtypeStruct((ROWS, 1), jnp.int32),
        scratch_shapes=[
            pltpu.VMEM((RB, 1), jnp.float32),
            pltpu.VMEM((RB, 1), jnp.int32),
            pltpu.VMEM((RB, 1), jnp.float32),
            pltpu.VMEM((RB, 1), jnp.int32),
        ],
        compiler_params=pltpu.CompilerParams(
            dimension_semantics=("parallel", "arbitrary"),
        ),
    )(logits, temps2d)
    return out.reshape(ROWS)
